# baseline (speedup 1.0000x reference)
import jax
import jax.numpy as jnp
from jax import lax
from jax.experimental import pallas as pl
from jax.experimental.pallas import tpu as pltpu
from jax.experimental.pallas import tpu_sc as plsc

N_CORES = 2
N_SUB = 16
NPAD = 10240
R = NPAD // N_SUB
CH = 64
STEPS = 10
K = 128
GROW = 8
ZR = 32


def _sc_body(src_hbm, dst_hbm, xd_hbm, xs_hbm, dis2_hbm, fs_hbm, a0_hbm,
             gam_hbm, out_hbm, p_hbm,
             srcl_v, dstl_v, rows_v, rows2_v, rows3_v, rows4_v, zero_v,
             dis2_v, fs_v, a0_v, gam_v, acc_sh, sem, sem2, sem3, sem4,
             sems1, sems2, sems3, sems4):
  c = lax.axis_index("c")
  s = lax.axis_index("s")
  lo = s * R
  base = c * NPAD + lo
  coff = c * NPAD
  nch = dstl_v.shape[0]

  pltpu.sync_copy(dis2_hbm.at[pl.ds(lo, R)], dis2_v)
  pltpu.sync_copy(fs_hbm.at[pl.ds(lo, R)], fs_v)
  pltpu.sync_copy(a0_hbm.at[pl.ds(lo, R)], a0_v)
  pltpu.sync_copy(gam_hbm, gam_v)

  pltpu.sync_copy(src_hbm.at[pl.ds(s * nch, nch)], srcl_v.at[pl.ds(0, nch)])
  pltpu.sync_copy(dst_hbm.at[pl.ds(s * nch, nch)], dstl_v)
  def bias_row(i, _):
    for g8 in range(K // 16):
      sl = pl.ds(g8 * 16, 16)
      srcl_v[i, sl] = srcl_v[i, sl] + coff
    return 0
  lax.fori_loop(0, nch, bias_row, 0)
  cof16 = jnp.full((16,), coff, jnp.int32)
  for g8 in range(K // 16):
    srcl_v[nch, pl.ds(g8 * 16, 16)] = cof16

  zv = jnp.zeros((16,), jnp.float32)
  def zrow(i, _):
    for q in range(CH // 16):
      zero_v[i, pl.ds(q * 16, 16)] = zv
    return 0
  lax.fori_loop(0, ZR, zrow, 0)
  def zacc(o, _):
    pltpu.sync_copy(zero_v, acc_sh.at[pl.ds(lo + o * ZR, ZR)])
    return 0
  lax.fori_loop(0, R // ZR, zacc, 0)
  @pl.when(s == 0)
  def _():
    pltpu.sync_copy(zero_v.at[pl.ds(0, GROW)], acc_sh.at[pl.ds(NPAD, GROW)])

  def gather(ch_i, rows_ref, sem_ref):
    pltpu.async_copy(p_hbm.at[srcl_v.at[ch_i]], rows_ref, sem_ref)

  def gwait(ch_i, rows_ref, sem_ref):
    pltpu.make_async_copy(p_hbm.at[srcl_v.at[ch_i]], rows_ref, sem_ref).wait()

  def scat(ch_i, rows_ref, sem_ref):
    pltpu.async_copy(rows_ref, acc_sh.at[dstl_v.at[ch_i]], sem_ref, add=True)

  def swait(ch_i, rows_ref, sem_ref):
    pltpu.make_async_copy(rows_ref, acc_sh.at[dstl_v.at[ch_i]],
                          sem_ref).wait()

  bufs = (rows_v, rows2_v, rows3_v, rows4_v)
  gsems = (sem, sem2, sem3, sem4)
  ssems = (sems1, sems2, sems3, sems4)

  def accumulate():
    for b in range(4):
      gather(jnp.int32(b), bufs[b], gsems[b])
    def quad_body(i, _):
      j0 = 4 * i
      for b in range(4):
        gwait(j0 + b, bufs[b], gsems[b])
        scat(j0 + b, bufs[b], ssems[b])
      for b in range(4):
        swait(j0 + b, bufs[b], ssems[b])
        gather(jnp.minimum(j0 + b + 4, nch), bufs[b], gsems[b])
      return 0
    lax.fori_loop(0, nch // 4, quad_body, 0)
    for b in range(4):
      gwait(jnp.int32(nch), bufs[b], gsems[b])

  lanes = lax.iota(jnp.int32, 16)

  def finalize(j):
    gvec = gam_v[pl.ds(0, 16)]
    g = jnp.sum(jnp.where(lanes == j, gvec, 0.0))
    for o in range(R // K):
      pltpu.sync_copy(acc_sh.at[pl.ds(lo + o * K, K)], rows_v)
      pltpu.sync_copy(xd_hbm.at[pl.ds(base + o * K, K)], rows3_v)
      def frg(rg, _):
        d2v = dis2_v[pl.ds(o * K + rg * 16, 16)]
        for r16 in range(16):
          r = rg * 16 + r16
          for q in range(CH // 16):
            sl = pl.ds(q * 16, 16)
            rows2_v[r, sl] = rows_v[r, sl] * d2v[r16] + rows3_v[r, sl] * g
        return 0
      lax.fori_loop(0, K // 16, frg, 0)
      pltpu.sync_copy(rows2_v, p_hbm.at[pl.ds(base + o * K, K)])
      for z in range(K // ZR):
        pltpu.sync_copy(zero_v, acc_sh.at[pl.ds(lo + o * K + z * ZR, ZR)])

  def step(t, _):
    plsc.subcore_barrier()
    @pl.when(t > 0)
    def _():
      accumulate()
      plsc.subcore_barrier()
    finalize(jnp.int32(STEPS) - t)
    return 0
  lax.fori_loop(0, STEPS + 1, step, 0)

  for o in range(R // K):
    pltpu.sync_copy(p_hbm.at[pl.ds(base + o * K, K)], rows3_v)
    pltpu.sync_copy(xs_hbm.at[pl.ds(base + o * K, K)], rows_v)
    def org(rg, _):
      fv = fs_v[pl.ds(o * K + rg * 16, 16)]
      av = a0_v[pl.ds(o * K + rg * 16, 16)]
      for r16 in range(16):
        r = rg * 16 + r16
        for q in range(CH // 16):
          sl = pl.ds(q * 16, 16)
          rows2_v[r, sl] = (rows3_v[r, sl] * fv[r16]
                            + rows_v[r, sl] * av[r16])
      return 0
    lax.fori_loop(0, K // 16, org, 0)
    pltpu.sync_copy(rows2_v, out_hbm.at[pl.ds(base + o * K, K)])


@jax.jit
def _gpr_sc(src2d, dst2d, xd_split, xs_split, dis2p, fsp, a0p, gamp):
  mesh = plsc.VectorSubcoreMesh(core_axis_name="c", subcore_axis_name="s",
                                num_cores=N_CORES, num_subcores=N_SUB)
  f32 = jnp.float32
  nch = src2d.shape[0] // N_SUB
  run = pl.kernel(
      _sc_body,
      out_type=(jax.ShapeDtypeStruct((N_CORES * NPAD, CH), f32),
                jax.ShapeDtypeStruct((N_CORES * NPAD, CH), f32)),
      mesh=mesh,
      compiler_params=pltpu.CompilerParams(
          use_tc_tiling_on_sc=False, needs_layout_passes=False),
      scratch_types=[
          pltpu.VMEM((nch + 1, K), jnp.int32),
          pltpu.VMEM((nch, K), jnp.int32),
          pltpu.VMEM((K, CH), f32),
          pltpu.VMEM((K, CH), f32),
          pltpu.VMEM((K, CH), f32),
          pltpu.VMEM((K, CH), f32),
          pltpu.VMEM((ZR, CH), f32),
          pltpu.VMEM((R,), f32),
          pltpu.VMEM((R,), f32),
          pltpu.VMEM((R,), f32),
          pltpu.VMEM((16,), f32),
          pltpu.VMEM_SHARED((NPAD + GROW, CH), f32),
          pltpu.SemaphoreType.DMA,
          pltpu.SemaphoreType.DMA,
          pltpu.SemaphoreType.DMA,
          pltpu.SemaphoreType.DMA,
          pltpu.SemaphoreType.DMA,
          pltpu.SemaphoreType.DMA,
          pltpu.SemaphoreType.DMA,
          pltpu.SemaphoreType.DMA,
      ],
  )
  return run(src2d, dst2d, xd_split, xs_split, dis2p, fsp, a0p, gamp)


def kernel(x, edge_index, gamma):
  n, ch = x.shape
  e = edge_index.shape[1]
  src = edge_index[0].astype(jnp.int32)
  dst = edge_index[1].astype(jnp.int32)

  deg = jnp.bincount(dst, length=n)
  degf = deg.astype(jnp.float32)
  pos = deg > 0
  dis = jnp.where(pos, lax.rsqrt(jnp.maximum(degf, 1.0)), 0.0)
  dis2 = dis * dis
  fs = jnp.where(pos, jnp.sqrt(degf), 0.0)
  a0 = jnp.where(pos, 0.0, gamma[0])

  pad = NPAD - n
  dis2p = jnp.pad(dis2, (0, pad))
  fsp = jnp.pad(fs, (0, pad))
  a0p = jnp.pad(a0, (0, pad))
  xd = dis[:, None] * x
  xdp = jnp.pad(xd, ((0, pad), (0, 0)))
  xsp = jnp.pad(x, ((0, pad), (0, 0)))
  xd_split = xdp.reshape(NPAD, N_CORES, CH).transpose(1, 0, 2).reshape(-1, CH)
  xs_split = xsp.reshape(NPAD, N_CORES, CH).transpose(1, 0, 2).reshape(-1, CH)

  grp = N_SUB * K
  nch = (e + grp - 1) // grp
  nch += (-nch) % 4
  ep = nch * grp
  src2d = jnp.pad(src, (0, ep - e)).reshape(-1, K)
  dst2d = jnp.pad(dst, (0, ep - e), constant_values=NPAD).reshape(-1, K)
  gamp = jnp.pad(gamma, (0, 16 - gamma.shape[0]))

  out_split, _ = _gpr_sc(src2d, dst2d, xd_split, xs_split, dis2p, fsp, a0p,
                         gamp)
  out = out_split.reshape(N_CORES, NPAD, CH)[:, :n, :]
  return out.transpose(1, 0, 2).reshape(n, ch)


# speedup vs baseline: 1.2672x; 1.2672x over previous
import jax
import jax.numpy as jnp
from jax import lax
from jax.experimental import pallas as pl
from jax.experimental.pallas import tpu as pltpu
from jax.experimental.pallas import tpu_sc as plsc

N_CORES = 2
N_SUB = 16
NPAD = 10240
R = NPAD // N_SUB
CH = 64
STEPS = 10
K = 128
GROW = 8
ZR = 32


def _sc_body(src_hbm, dst_hbm, xd_hbm, xs_hbm, dis2_hbm, fs_hbm, a0_hbm,
             gam_hbm, out_hbm, p_hbm,
             srcl_v, dstl_v, rows_v, rows2_v, rows3_v, rows4_v, zero_v,
             dis2_v, fs_v, a0_v, gam_v, acc_sh, sem, sem2, sem3):
  c = lax.axis_index("c")
  s = lax.axis_index("s")
  lo = s * R
  base = c * NPAD + lo
  coff = c * NPAD
  nch = dstl_v.shape[0]

  pltpu.sync_copy(dis2_hbm.at[pl.ds(lo, R)], dis2_v)
  pltpu.sync_copy(fs_hbm.at[pl.ds(lo, R)], fs_v)
  pltpu.sync_copy(a0_hbm.at[pl.ds(lo, R)], a0_v)
  pltpu.sync_copy(gam_hbm, gam_v)

  pltpu.sync_copy(src_hbm.at[pl.ds(s * nch, nch)], srcl_v.at[pl.ds(0, nch)])
  pltpu.sync_copy(dst_hbm.at[pl.ds(s * nch, nch)], dstl_v)
  def bias_row(i, _):
    for g8 in range(K // 16):
      sl = pl.ds(g8 * 16, 16)
      srcl_v[i, sl] = srcl_v[i, sl] + coff
    return 0
  lax.fori_loop(0, nch, bias_row, 0)
  cof16 = jnp.full((16,), coff, jnp.int32)
  for g8 in range(K // 16):
    srcl_v[nch, pl.ds(g8 * 16, 16)] = cof16

  zv = jnp.zeros((16,), jnp.float32)
  def zrow(i, _):
    for q in range(CH // 16):
      zero_v[i, pl.ds(q * 16, 16)] = zv
    return 0
  lax.fori_loop(0, ZR, zrow, 0)
  def zacc(o, _):
    pltpu.sync_copy(zero_v, acc_sh.at[pl.ds(lo + o * ZR, ZR)])
    return 0
  lax.fori_loop(0, R // ZR, zacc, 0)
  @pl.when(s == 0)
  def _():
    pltpu.sync_copy(zero_v.at[pl.ds(0, GROW)], acc_sh.at[pl.ds(NPAD, GROW)])

  def gather(ch_i, rows_ref, sem_ref):
    pltpu.async_copy(p_hbm.at[srcl_v.at[ch_i]], rows_ref, sem_ref)

  def gwait(ch_i, rows_ref, sem_ref):
    pltpu.make_async_copy(p_hbm.at[srcl_v.at[ch_i]], rows_ref, sem_ref).wait()

  def scat(ch_i, rows_ref):
    pltpu.sync_copy(rows_ref, acc_sh.at[dstl_v.at[ch_i]], add=True)

  bufs = (rows_v, rows2_v, rows4_v)
  gsems = (sem, sem2, sem3)

  def accumulate():
    for b in range(3):
      gather(jnp.int32(b), bufs[b], gsems[b])
    def tri_body(i, _):
      j0 = 3 * i
      for b in range(3):
        gwait(j0 + b, bufs[b], gsems[b])
        scat(j0 + b, bufs[b])
        gather(jnp.minimum(j0 + b + 3, nch), bufs[b], gsems[b])
      return 0
    lax.fori_loop(0, nch // 3, tri_body, 0)
    for b in range(3):
      gwait(jnp.int32(nch), bufs[b], gsems[b])

  lanes = lax.iota(jnp.int32, 16)

  def finalize(j):
    gvec = gam_v[pl.ds(0, 16)]
    g = jnp.sum(jnp.where(lanes == j, gvec, 0.0))
    for o in range(R // K):
      pltpu.sync_copy(acc_sh.at[pl.ds(lo + o * K, K)], rows_v)
      pltpu.sync_copy(xd_hbm.at[pl.ds(base + o * K, K)], rows3_v)
      def frg(rg, _):
        d2v = dis2_v[pl.ds(o * K + rg * 16, 16)]
        for r16 in range(16):
          r = rg * 16 + r16
          for q in range(CH // 16):
            sl = pl.ds(q * 16, 16)
            rows2_v[r, sl] = rows_v[r, sl] * d2v[r16] + rows3_v[r, sl] * g
        return 0
      lax.fori_loop(0, K // 16, frg, 0)
      pltpu.sync_copy(rows2_v, p_hbm.at[pl.ds(base + o * K, K)])
      for z in range(K // ZR):
        pltpu.sync_copy(zero_v, acc_sh.at[pl.ds(lo + o * K + z * ZR, ZR)])

  def step(t, _):
    plsc.subcore_barrier()
    @pl.when(t > 0)
    def _():
      accumulate()
      plsc.subcore_barrier()
    finalize(jnp.int32(STEPS) - t)
    return 0
  lax.fori_loop(0, STEPS + 1, step, 0)

  for o in range(R // K):
    pltpu.sync_copy(p_hbm.at[pl.ds(base + o * K, K)], rows3_v)
    pltpu.sync_copy(xs_hbm.at[pl.ds(base + o * K, K)], rows_v)
    def org(rg, _):
      fv = fs_v[pl.ds(o * K + rg * 16, 16)]
      av = a0_v[pl.ds(o * K + rg * 16, 16)]
      for r16 in range(16):
        r = rg * 16 + r16
        for q in range(CH // 16):
          sl = pl.ds(q * 16, 16)
          rows2_v[r, sl] = (rows3_v[r, sl] * fv[r16]
                            + rows_v[r, sl] * av[r16])
      return 0
    lax.fori_loop(0, K // 16, org, 0)
    pltpu.sync_copy(rows2_v, out_hbm.at[pl.ds(base + o * K, K)])


@jax.jit
def _gpr_sc(src2d, dst2d, xd_split, xs_split, dis2p, fsp, a0p, gamp):
  mesh = plsc.VectorSubcoreMesh(core_axis_name="c", subcore_axis_name="s",
                                num_cores=N_CORES, num_subcores=N_SUB)
  f32 = jnp.float32
  nch = src2d.shape[0] // N_SUB
  run = pl.kernel(
      _sc_body,
      out_type=(jax.ShapeDtypeStruct((N_CORES * NPAD, CH), f32),
                jax.ShapeDtypeStruct((N_CORES * NPAD, CH), f32)),
      mesh=mesh,
      compiler_params=pltpu.CompilerParams(
          use_tc_tiling_on_sc=False, needs_layout_passes=False),
      scratch_types=[
          pltpu.VMEM((nch + 1, K), jnp.int32),
          pltpu.VMEM((nch, K), jnp.int32),
          pltpu.VMEM((K, CH), f32),
          pltpu.VMEM((K, CH), f32),
          pltpu.VMEM((K, CH), f32),
          pltpu.VMEM((K, CH), f32),
          pltpu.VMEM((ZR, CH), f32),
          pltpu.VMEM((R,), f32),
          pltpu.VMEM((R,), f32),
          pltpu.VMEM((R,), f32),
          pltpu.VMEM((16,), f32),
          pltpu.VMEM_SHARED((NPAD + GROW, CH), f32),
          pltpu.SemaphoreType.DMA,
          pltpu.SemaphoreType.DMA,
          pltpu.SemaphoreType.DMA,
      ],
  )
  return run(src2d, dst2d, xd_split, xs_split, dis2p, fsp, a0p, gamp)


def kernel(x, edge_index, gamma):
  n, ch = x.shape
  e = edge_index.shape[1]
  src = edge_index[0].astype(jnp.int32)
  dst = edge_index[1].astype(jnp.int32)

  deg = jnp.bincount(dst, length=n)
  degf = deg.astype(jnp.float32)
  pos = deg > 0
  dis = jnp.where(pos, lax.rsqrt(jnp.maximum(degf, 1.0)), 0.0)
  dis2 = dis * dis
  fs = jnp.where(pos, jnp.sqrt(degf), 0.0)
  a0 = jnp.where(pos, 0.0, gamma[0])

  pad = NPAD - n
  dis2p = jnp.pad(dis2, (0, pad))
  fsp = jnp.pad(fs, (0, pad))
  a0p = jnp.pad(a0, (0, pad))
  xd = dis[:, None] * x
  xdp = jnp.pad(xd, ((0, pad), (0, 0)))
  xsp = jnp.pad(x, ((0, pad), (0, 0)))
  xd_split = xdp.reshape(NPAD, N_CORES, CH).transpose(1, 0, 2).reshape(-1, CH)
  xs_split = xsp.reshape(NPAD, N_CORES, CH).transpose(1, 0, 2).reshape(-1, CH)

  grp = N_SUB * K
  nch = (e + grp - 1) // grp
  nch += (-nch) % 3
  ep = nch * grp
  src2d = jnp.pad(src, (0, ep - e)).reshape(-1, K)
  dst2d = jnp.pad(dst, (0, ep - e), constant_values=NPAD).reshape(-1, K)
  gamp = jnp.pad(gamma, (0, 16 - gamma.shape[0]))

  out_split, _ = _gpr_sc(src2d, dst2d, xd_split, xs_split, dis2p, fsp, a0p,
                         gamp)
  out = out_split.reshape(N_CORES, NPAD, CH)[:, :n, :]
  return out.transpose(1, 0, 2).reshape(n, ch)


# speedup vs baseline: 1.7653x; 1.3931x over previous
import jax
import jax.numpy as jnp
from jax import lax
from jax.experimental import pallas as pl
from jax.experimental.pallas import tpu as pltpu
from jax.experimental.pallas import tpu_sc as plsc

N_CORES = 2
N_SUB = 16
NPAD = 10240
R = NPAD // N_SUB
CH = 64
STEPS = 10
K = 128
GROW = 8
ZR = 32


def _sc_body(src_hbm, dst_hbm, xd_hbm, xs_hbm, dis2_hbm, fs_hbm, a0_hbm,
             gam_hbm, out_hbm, p_hbm,
             srcl_v, dstl_v, rows_v, rows2_v, rows3_v, rows4_v, zero_v,
             dis2_v, fs_v, a0_v, gam_v, acc_sh, sem, sem2, sem3):
  c = lax.axis_index("c")
  s = lax.axis_index("s")
  lo = s * R
  base = c * NPAD + lo
  coff = c * NPAD
  nch = dstl_v.shape[0]

  pltpu.sync_copy(dis2_hbm.at[pl.ds(lo, R)], dis2_v)
  pltpu.sync_copy(fs_hbm.at[pl.ds(lo, R)], fs_v)
  pltpu.sync_copy(a0_hbm.at[pl.ds(lo, R)], a0_v)
  pltpu.sync_copy(gam_hbm, gam_v)

  pltpu.sync_copy(src_hbm.at[pl.ds(s * nch, nch)], srcl_v.at[pl.ds(0, nch)])
  pltpu.sync_copy(dst_hbm.at[pl.ds(s * nch, nch)], dstl_v)
  def bias_row(i, _):
    for g8 in range(K // 16):
      sl = pl.ds(g8 * 16, 16)
      srcl_v[i, sl] = srcl_v[i, sl] + coff
    return 0
  lax.fori_loop(0, nch, bias_row, 0)
  cof16 = jnp.full((16,), coff, jnp.int32)
  for g8 in range(K // 16):
    srcl_v[nch, pl.ds(g8 * 16, 16)] = cof16

  zv = jnp.zeros((16,), jnp.float32)
  def zrow(i, _):
    for q in range(CH // 16):
      zero_v[i, pl.ds(q * 16, 16)] = zv
    return 0
  lax.fori_loop(0, ZR, zrow, 0)
  def zacc(o, _):
    pltpu.sync_copy(zero_v, acc_sh.at[pl.ds(lo + o * ZR, ZR)])
    return 0
  lax.fori_loop(0, R // ZR, zacc, 0)
  @pl.when(s == 0)
  def _():
    pltpu.sync_copy(zero_v.at[pl.ds(0, GROW)], acc_sh.at[pl.ds(NPAD, GROW)])

  def gather(ch_i, rows_ref, sem_ref):
    pltpu.async_copy(p_hbm.at[srcl_v.at[ch_i]], rows_ref, sem_ref)

  def gwait(ch_i, rows_ref, sem_ref):
    pltpu.make_async_copy(p_hbm.at[srcl_v.at[ch_i]], rows_ref, sem_ref).wait()

  def scat(ch_i, rows_ref):
    pltpu.sync_copy(rows_ref, acc_sh.at[dstl_v.at[ch_i]], add=True)

  def accumulate():
    gather(jnp.int32(0), rows_v, sem)
    def pair_body(i, _):
      c0 = 2 * i
      gather(c0 + 1, rows2_v, sem2)
      gwait(c0, rows_v, sem)
      scat(c0, rows_v)
      gather(c0 + 2, rows_v, sem)
      gwait(c0 + 1, rows2_v, sem2)
      scat(c0 + 1, rows2_v)
      return 0
    lax.fori_loop(0, nch // 2, pair_body, 0)
    gwait(jnp.int32(nch), rows_v, sem)

  lanes = lax.iota(jnp.int32, 16)

  def finalize(j):
    gvec = gam_v[pl.ds(0, 16)]
    g = jnp.sum(jnp.where(lanes == j, gvec, 0.0))
    for o in range(R // K):
      pltpu.sync_copy(acc_sh.at[pl.ds(lo + o * K, K)], rows_v)
      pltpu.sync_copy(xd_hbm.at[pl.ds(base + o * K, K)], rows3_v)
      def frg(rg, _):
        d2v = dis2_v[pl.ds(o * K + rg * 16, 16)]
        for r16 in range(16):
          r = rg * 16 + r16
          for q in range(CH // 16):
            sl = pl.ds(q * 16, 16)
            rows2_v[r, sl] = rows_v[r, sl] * d2v[r16] + rows3_v[r, sl] * g
        return 0
      lax.fori_loop(0, K // 16, frg, 0)
      pltpu.sync_copy(rows2_v, p_hbm.at[pl.ds(base + o * K, K)])
      for z in range(K // ZR):
        pltpu.sync_copy(zero_v, acc_sh.at[pl.ds(lo + o * K + z * ZR, ZR)])

  def step(t, _):
    plsc.subcore_barrier()
    @pl.when(t > 0)
    def _():
      accumulate()
      plsc.subcore_barrier()
    finalize(jnp.int32(STEPS) - t)
    return 0
  lax.fori_loop(0, STEPS + 1, step, 0)

  for o in range(R // K):
    pltpu.sync_copy(p_hbm.at[pl.ds(base + o * K, K)], rows3_v)
    pltpu.sync_copy(xs_hbm.at[pl.ds(base + o * K, K)], rows_v)
    def org(rg, _):
      fv = fs_v[pl.ds(o * K + rg * 16, 16)]
      av = a0_v[pl.ds(o * K + rg * 16, 16)]
      for r16 in range(16):
        r = rg * 16 + r16
        for q in range(CH // 16):
          sl = pl.ds(q * 16, 16)
          rows2_v[r, sl] = (rows3_v[r, sl] * fv[r16]
                            + rows_v[r, sl] * av[r16])
      return 0
    lax.fori_loop(0, K // 16, org, 0)
    pltpu.sync_copy(rows2_v, out_hbm.at[pl.ds(base + o * K, K)])


@jax.jit
def _gpr_sc(src2d, dst2d, xd_split, xs_split, dis2p, fsp, a0p, gamp):
  mesh = plsc.VectorSubcoreMesh(core_axis_name="c", subcore_axis_name="s",
                                num_cores=N_CORES, num_subcores=N_SUB)
  f32 = jnp.float32
  nch = src2d.shape[0] // N_SUB
  run = pl.kernel(
      _sc_body,
      out_type=(jax.ShapeDtypeStruct((N_CORES * NPAD, CH), f32),
                jax.ShapeDtypeStruct((N_CORES * NPAD, CH), f32)),
      mesh=mesh,
      compiler_params=pltpu.CompilerParams(
          use_tc_tiling_on_sc=False, needs_layout_passes=False),
      scratch_types=[
          pltpu.VMEM((nch + 1, K), jnp.int32),
          pltpu.VMEM((nch, K), jnp.int32),
          pltpu.VMEM((K, CH), f32),
          pltpu.VMEM((K, CH), f32),
          pltpu.VMEM((K, CH), f32),
          pltpu.VMEM((K, CH), f32),
          pltpu.VMEM((ZR, CH), f32),
          pltpu.VMEM((R,), f32),
          pltpu.VMEM((R,), f32),
          pltpu.VMEM((R,), f32),
          pltpu.VMEM((16,), f32),
          pltpu.VMEM_SHARED((NPAD + GROW, CH), f32),
          pltpu.SemaphoreType.DMA,
          pltpu.SemaphoreType.DMA,
          pltpu.SemaphoreType.DMA,
      ],
  )
  return run(src2d, dst2d, xd_split, xs_split, dis2p, fsp, a0p, gamp)


def kernel(x, edge_index, gamma):
  n, ch = x.shape
  e = edge_index.shape[1]
  src = edge_index[0].astype(jnp.int32)
  dst = edge_index[1].astype(jnp.int32)

  deg = jnp.bincount(dst, length=n)
  degf = deg.astype(jnp.float32)
  pos = deg > 0
  dis = jnp.where(pos, lax.rsqrt(jnp.maximum(degf, 1.0)), 0.0)
  dis2 = dis * dis
  fs = jnp.where(pos, jnp.sqrt(degf), 0.0)
  a0 = jnp.where(pos, 0.0, gamma[0])

  pad = NPAD - n
  dis2p = jnp.pad(dis2, (0, pad))
  fsp = jnp.pad(fs, (0, pad))
  a0p = jnp.pad(a0, (0, pad))
  xd = dis[:, None] * x
  xdp = jnp.pad(xd, ((0, pad), (0, 0)))
  xsp = jnp.pad(x, ((0, pad), (0, 0)))
  xd_split = xdp.reshape(NPAD, N_CORES, CH).transpose(1, 0, 2).reshape(-1, CH)
  xs_split = xsp.reshape(NPAD, N_CORES, CH).transpose(1, 0, 2).reshape(-1, CH)

  grp = N_SUB * K
  nch = (e + grp - 1) // grp
  nch += nch % 2
  ep = nch * grp
  src2d = jnp.pad(src, (0, ep - e)).reshape(-1, K)
  dst2d = jnp.pad(dst, (0, ep - e), constant_values=NPAD).reshape(-1, K)
  gamp = jnp.pad(gamma, (0, 16 - gamma.shape[0]))

  out_split, _ = _gpr_sc(src2d, dst2d, xd_split, xs_split, dis2p, fsp, a0p,
                         gamp)
  out = out_split.reshape(N_CORES, NPAD, CH)[:, :n, :]
  return out.transpose(1, 0, 2).reshape(n, ch)


# speedup vs baseline: 1.7684x; 1.0017x over previous
import jax
import jax.numpy as jnp
from jax import lax
from jax.experimental import pallas as pl
from jax.experimental.pallas import tpu as pltpu
from jax.experimental.pallas import tpu_sc as plsc

N_CORES = 2
N_SUB = 16
NPAD = 10240
R = NPAD // N_SUB
CH = 64
STEPS = 10
K = 128
GROW = 8
ZR = 128


def _sc_body(src_hbm, dst_hbm, xd_hbm, xs_hbm, dis2_hbm, fs_hbm, a0_hbm,
             gam_hbm, out_hbm, p_hbm,
             srcl_v, dstl_v, rows_v, rows2_v, rows3_v, rows4_v, zero_v,
             dis2_v, fs_v, a0_v, gam_v, acc_sh, sem, sem2, sem3):
  c = lax.axis_index("c")
  s = lax.axis_index("s")
  lo = s * R
  base = c * NPAD + lo
  coff = c * NPAD
  nch = dstl_v.shape[0]

  pltpu.sync_copy(dis2_hbm.at[pl.ds(lo, R)], dis2_v)
  pltpu.sync_copy(fs_hbm.at[pl.ds(lo, R)], fs_v)
  pltpu.sync_copy(a0_hbm.at[pl.ds(lo, R)], a0_v)
  pltpu.sync_copy(gam_hbm, gam_v)

  pltpu.sync_copy(src_hbm.at[pl.ds(s * nch, nch)], srcl_v.at[pl.ds(0, nch)])
  pltpu.sync_copy(dst_hbm.at[pl.ds(s * nch, nch)], dstl_v)
  def bias_row(i, _):
    for g8 in range(K // 16):
      sl = pl.ds(g8 * 16, 16)
      srcl_v[i, sl] = srcl_v[i, sl] + coff
    return 0
  lax.fori_loop(0, nch, bias_row, 0)
  cof16 = jnp.full((16,), coff, jnp.int32)
  for g8 in range(K // 16):
    srcl_v[nch, pl.ds(g8 * 16, 16)] = cof16

  zv = jnp.zeros((16,), jnp.float32)
  def zrow(i, _):
    for q in range(CH // 16):
      zero_v[i, pl.ds(q * 16, 16)] = zv
    return 0
  lax.fori_loop(0, ZR, zrow, 0)
  def zacc(o, _):
    pltpu.sync_copy(zero_v, acc_sh.at[pl.ds(lo + o * ZR, ZR)])
    return 0
  lax.fori_loop(0, R // ZR, zacc, 0)
  @pl.when(s == 0)
  def _():
    pltpu.sync_copy(zero_v.at[pl.ds(0, GROW)], acc_sh.at[pl.ds(NPAD, GROW)])

  def gather(ch_i, rows_ref, sem_ref):
    pltpu.async_copy(p_hbm.at[srcl_v.at[ch_i]], rows_ref, sem_ref)

  def gwait(ch_i, rows_ref, sem_ref):
    pltpu.make_async_copy(p_hbm.at[srcl_v.at[ch_i]], rows_ref, sem_ref).wait()

  def scat(ch_i, rows_ref):
    pltpu.sync_copy(rows_ref, acc_sh.at[dstl_v.at[ch_i]], add=True)

  def accumulate():
    gather(jnp.int32(0), rows_v, sem)
    def pair_body(i, _):
      c0 = 2 * i
      gather(c0 + 1, rows2_v, sem2)
      gwait(c0, rows_v, sem)
      scat(c0, rows_v)
      gather(c0 + 2, rows_v, sem)
      gwait(c0 + 1, rows2_v, sem2)
      scat(c0 + 1, rows2_v)
      return 0
    lax.fori_loop(0, nch // 2, pair_body, 0)
    gwait(jnp.int32(nch), rows_v, sem)

  lanes = lax.iota(jnp.int32, 16)

  def finalize(j):
    gvec = gam_v[pl.ds(0, 16)]
    g = jnp.sum(jnp.where(lanes == j, gvec, 0.0))
    for o in range(R // K):
      pltpu.sync_copy(acc_sh.at[pl.ds(lo + o * K, K)], rows_v)
      pltpu.sync_copy(xd_hbm.at[pl.ds(base + o * K, K)], rows3_v)
      def frg(rg, _):
        d2v = dis2_v[pl.ds(o * K + rg * 16, 16)]
        for r16 in range(16):
          r = rg * 16 + r16
          for q in range(CH // 16):
            sl = pl.ds(q * 16, 16)
            rows2_v[r, sl] = rows_v[r, sl] * d2v[r16] + rows3_v[r, sl] * g
        return 0
      lax.fori_loop(0, K // 16, frg, 0)
      pltpu.sync_copy(rows2_v, p_hbm.at[pl.ds(base + o * K, K)])
      for z in range(K // ZR):
        pltpu.sync_copy(zero_v, acc_sh.at[pl.ds(lo + o * K + z * ZR, ZR)])

  def step(t, _):
    plsc.subcore_barrier()
    @pl.when(t > 0)
    def _():
      accumulate()
      plsc.subcore_barrier()
    finalize(jnp.int32(STEPS) - t)
    return 0
  lax.fori_loop(0, STEPS + 1, step, 0)

  for o in range(R // K):
    pltpu.sync_copy(p_hbm.at[pl.ds(base + o * K, K)], rows3_v)
    pltpu.sync_copy(xs_hbm.at[pl.ds(base + o * K, K)], rows_v)
    def org(rg, _):
      fv = fs_v[pl.ds(o * K + rg * 16, 16)]
      av = a0_v[pl.ds(o * K + rg * 16, 16)]
      for r16 in range(16):
        r = rg * 16 + r16
        for q in range(CH // 16):
          sl = pl.ds(q * 16, 16)
          rows2_v[r, sl] = (rows3_v[r, sl] * fv[r16]
                            + rows_v[r, sl] * av[r16])
      return 0
    lax.fori_loop(0, K // 16, org, 0)
    pltpu.sync_copy(rows2_v, out_hbm.at[pl.ds(base + o * K, K)])


@jax.jit
def _gpr_sc(src2d, dst2d, xd_split, xs_split, dis2p, fsp, a0p, gamp):
  mesh = plsc.VectorSubcoreMesh(core_axis_name="c", subcore_axis_name="s",
                                num_cores=N_CORES, num_subcores=N_SUB)
  f32 = jnp.float32
  nch = src2d.shape[0] // N_SUB
  run = pl.kernel(
      _sc_body,
      out_type=(jax.ShapeDtypeStruct((N_CORES * NPAD, CH), f32),
                jax.ShapeDtypeStruct((N_CORES * NPAD, CH), f32)),
      mesh=mesh,
      compiler_params=pltpu.CompilerParams(
          use_tc_tiling_on_sc=False, needs_layout_passes=False),
      scratch_types=[
          pltpu.VMEM((nch + 1, K), jnp.int32),
          pltpu.VMEM((nch, K), jnp.int32),
          pltpu.VMEM((K, CH), f32),
          pltpu.VMEM((K, CH), f32),
          pltpu.VMEM((K, CH), f32),
          pltpu.VMEM((K, CH), f32),
          pltpu.VMEM((ZR, CH), f32),
          pltpu.VMEM((R,), f32),
          pltpu.VMEM((R,), f32),
          pltpu.VMEM((R,), f32),
          pltpu.VMEM((16,), f32),
          pltpu.VMEM_SHARED((NPAD + GROW, CH), f32),
          pltpu.SemaphoreType.DMA,
          pltpu.SemaphoreType.DMA,
          pltpu.SemaphoreType.DMA,
      ],
  )
  return run(src2d, dst2d, xd_split, xs_split, dis2p, fsp, a0p, gamp)


def kernel(x, edge_index, gamma):
  n, ch = x.shape
  e = edge_index.shape[1]
  src = edge_index[0].astype(jnp.int32)
  dst = edge_index[1].astype(jnp.int32)

  deg = jnp.bincount(dst, length=n)
  degf = deg.astype(jnp.float32)
  pos = deg > 0
  dis = jnp.where(pos, lax.rsqrt(jnp.maximum(degf, 1.0)), 0.0)
  dis2 = dis * dis
  fs = jnp.where(pos, jnp.sqrt(degf), 0.0)
  a0 = jnp.where(pos, 0.0, gamma[0])

  pad = NPAD - n
  dis2p = jnp.pad(dis2, (0, pad))
  fsp = jnp.pad(fs, (0, pad))
  a0p = jnp.pad(a0, (0, pad))
  xd = dis[:, None] * x
  xdp = jnp.pad(xd, ((0, pad), (0, 0)))
  xsp = jnp.pad(x, ((0, pad), (0, 0)))
  xd_split = xdp.reshape(NPAD, N_CORES, CH).transpose(1, 0, 2).reshape(-1, CH)
  xs_split = xsp.reshape(NPAD, N_CORES, CH).transpose(1, 0, 2).reshape(-1, CH)

  grp = N_SUB * K
  nch = (e + grp - 1) // grp
  nch += nch % 2
  ep = nch * grp
  src2d = jnp.pad(src, (0, ep - e)).reshape(-1, K)
  dst2d = jnp.pad(dst, (0, ep - e), constant_values=NPAD).reshape(-1, K)
  gamp = jnp.pad(gamma, (0, 16 - gamma.shape[0]))

  out_split, _ = _gpr_sc(src2d, dst2d, xd_split, xs_split, dis2p, fsp, a0p,
                         gamp)
  out = out_split.reshape(N_CORES, NPAD, CH)[:, :n, :]
  return out.transpose(1, 0, 2).reshape(n, ch)
